# baseline (speedup 1.0000x reference)
import functools

import jax
import jax.numpy as jnp
import numpy as np
from jax import lax
from jax.experimental import pallas as pl
from jax.experimental.pallas import tpu as pltpu
from jax.experimental.pallas import tpu_sc as plsc

_OMEGA = 0.01
_K_FRAC = 0.5
_BM = 400
_NSTREAM = 5
_NC, _NS = 2, 16
_NW = _NC * _NS
_HI = jax.lax.Precision.HIGHEST


def _leaky(v):
    return jnp.where(v >= 0.0, v, v * 0.01)


def _h_body(x_ref, w_ref, b_ref, h_ref, *, scale):
    h_ref[...] = (jnp.dot(x_ref[...], w_ref[...],
                          preferred_element_type=jnp.float32)
                  + b_ref[...]) * scale


def _main_body(*refs, n_nodes):
    adj_refs = refs[:_NSTREAM]
    (hfull_ref, degc_ref, degr_ref, pe_ref, wg_ref, wb_ref, bg_ref, bb_ref,
     wa_ref, wr_ref, out_ref, lb_ref, lf_ref) = refs[_NSTREAM:]
    agg = jnp.concatenate(
        [jnp.dot(a[...], hfull_ref[...], preferred_element_type=jnp.float32)
         for a in adj_refs], axis=0)
    step = pl.program_id(0)
    degc = degc_ref[...]
    kthr = jnp.sum(degr_ref[...]) * (_K_FRAC / n_nodes)

    pe64 = pe_ref[0:64, :]
    g_mat = jnp.dot(pe64, wg_ref[...], precision=_HI,
                    preferred_element_type=jnp.float32)
    b_mat = jnp.dot(pe64, wb_ref[...], precision=_HI,
                    preferred_element_type=jnp.float32)
    ids = lax.broadcasted_iota(jnp.int32, (degc.shape[0], 64), 1)
    onehot = (ids == degc.astype(jnp.int32)).astype(jnp.float32)
    gamma = _leaky(jnp.dot(onehot, g_mat, precision=_HI,
                           preferred_element_type=jnp.float32) + bg_ref[...])
    beta = _leaky(jnp.dot(onehot, b_mat, precision=_HI,
                          preferred_element_type=jnp.float32) + bb_ref[...])

    deg_safe = jnp.where(degc == 0.0, 1.0, degc)
    ivec = jnp.where(degc == 0.0, 0.0, agg / deg_safe)
    gp1 = gamma + 1.0
    b_add = gp1 * jnp.dot(ivec, wa_ref[...], precision=_HI,
                          preferred_element_type=jnp.float32) + beta
    b_rev = gp1 * jnp.dot(ivec, wr_ref[...], precision=_HI,
                          preferred_element_type=jnp.float32) + beta
    r = (degc < kthr).astype(jnp.float32)
    bias = _OMEGA * (r * b_add - (1.0 - r) * b_rev)
    bm = degc.shape[0]
    h_blk = hfull_ref[pl.ds(step * bm, bm), :]
    out_ref[...] = (agg + h_blk + bias) / (degc + 1.0)

    n_add = jnp.sqrt(jnp.sum(b_add * b_add, axis=1, keepdims=True))
    n_rev = jnp.sqrt(jnp.sum(b_rev * b_rev, axis=1, keepdims=True))
    lb_ref[...] = r * n_add + (1.0 - r) * n_rev
    lf_ref[...] = (jnp.sqrt(jnp.sum(gamma * gamma, axis=1, keepdims=True))
                   + jnp.sqrt(jnp.sum(beta * beta, axis=1, keepdims=True)))


def _sc_loss_body(lb_hbm, lf_hbm, idx_hbm, out_lb, out_lf,
                  idx_v, lb_v, lf_v, acc_v, sem, *, b_per_w, n_valid):
    wid = lax.axis_index("s") * _NC + lax.axis_index("c")
    base = wid * b_per_w
    pltpu.sync_copy(idx_hbm.at[pl.ds(base, b_per_w)], idx_v)
    cp_lb = pltpu.async_copy(lb_hbm.at[idx_v], lb_v, sem)
    cp_lf = pltpu.async_copy(lf_hbm.at[idx_v], lf_v, sem)
    cp_lb.wait()
    cp_lf.wait()
    acc_lb = jnp.zeros((16,), jnp.float32)
    acc_lf = jnp.zeros((16,), jnp.float32)
    lanes = lax.iota(jnp.int32, 16)
    for j in range(b_per_w // 16):
        pos = base + (j * 16) + lanes
        mask = pos < n_valid
        acc_lb = acc_lb + jnp.where(mask, lb_v[pl.ds(j * 16, 16)], 0.0)
        acc_lf = acc_lf + jnp.where(mask, lf_v[pl.ds(j * 16, 16)], 0.0)
    acc_v[...] = acc_lb
    pltpu.sync_copy(acc_v, out_lb.at[wid])
    acc_v[...] = acc_lf
    pltpu.sync_copy(acc_v, out_lf.at[wid])


def kernel(x, adj, degree, idx, edge, W, b, W_gamma, W_beta, b_gamma, b_beta,
           W_add, W_rev, PE):
    n, d_in = x.shape
    d_out = W.shape[1]
    dim_m = PE.shape[1]
    scale = float(np.sqrt(dim_m))
    n_idx = idx.shape[0]

    b2 = b.reshape(1, d_out)
    degf = degree.astype(jnp.float32)
    degr = degf.reshape(1, n)

    h = pl.pallas_call(
        functools.partial(_h_body, scale=scale),
        out_shape=jax.ShapeDtypeStruct((n, d_out), jnp.float32),
    )(x, W, b2)

    nblk = n // _BM
    blk = lambda i: (i, 0)
    full = lambda i: (0, 0)
    out, lb_pn, lf_pn = pl.pallas_call(
        functools.partial(_main_body, n_nodes=n),
        grid=(nblk,),
        in_specs=[
            *[pl.BlockSpec((_BM // _NSTREAM, n),
                           functools.partial(
                               lambda i, s: (_NSTREAM * i + s, 0), s=s))
              for s in range(_NSTREAM)],
            pl.BlockSpec((n, d_out), full),
            pl.BlockSpec((_BM, 1), blk),
            pl.BlockSpec((1, n), full),
            pl.BlockSpec(PE.shape, full),
            pl.BlockSpec(W_gamma.shape, full),
            pl.BlockSpec(W_beta.shape, full),
            pl.BlockSpec(b_gamma.shape, full),
            pl.BlockSpec(b_beta.shape, full),
            pl.BlockSpec(W_add.shape, full),
            pl.BlockSpec(W_rev.shape, full),
        ],
        out_specs=[
            pl.BlockSpec((_BM, d_out), blk),
            pl.BlockSpec((_BM, 1), blk),
            pl.BlockSpec((_BM, 1), blk),
        ],
        out_shape=[
            jax.ShapeDtypeStruct((n, d_out), jnp.float32),
            jax.ShapeDtypeStruct((n, 1), jnp.float32),
            jax.ShapeDtypeStruct((n, 1), jnp.float32),
        ],
        compiler_params=pltpu.CompilerParams(
            dimension_semantics=("arbitrary",)),
    )(*([adj] * _NSTREAM), h, degf, degr, PE, W_gamma, W_beta, b_gamma,
      b_beta, W_add, W_rev)

    b_per_w = -(-n_idx // _NW)
    b_per_w = ((b_per_w + 15) // 16) * 16
    n_pad = b_per_w * _NW
    idx_pad = jnp.concatenate(
        [idx.astype(jnp.int32), jnp.zeros((n_pad - n_idx,), jnp.int32)])

    mesh = plsc.VectorSubcoreMesh(core_axis_name="c", subcore_axis_name="s")
    sc_loss = functools.partial(
        pl.kernel,
        mesh=mesh,
        out_type=[
            jax.ShapeDtypeStruct((_NW, 16), jnp.float32),
            jax.ShapeDtypeStruct((_NW, 16), jnp.float32),
        ],
        scratch_types=[
            pltpu.VMEM((b_per_w,), jnp.int32),
            pltpu.VMEM((b_per_w,), jnp.float32),
            pltpu.VMEM((b_per_w,), jnp.float32),
            pltpu.VMEM((16,), jnp.float32),
            pltpu.SemaphoreType.DMA,
        ],
        compiler_params=pltpu.CompilerParams(needs_layout_passes=False),
    )(functools.partial(_sc_loss_body, b_per_w=b_per_w, n_valid=n_idx))
    plb, plf = sc_loss(lb_pn.reshape(n), lf_pn.reshape(n), idx_pad)

    inv = 1.0 / n_idx
    l_b = jnp.sum(plb) * inv
    l_film = jnp.sum(plf) * inv
    return out, l_b, l_film


# speedup vs baseline: 1.0483x; 1.0483x over previous
import functools

import jax
import jax.numpy as jnp
import numpy as np
from jax import lax
from jax.experimental import pallas as pl
from jax.experimental.pallas import tpu as pltpu
from jax.experimental.pallas import tpu_sc as plsc

_OMEGA = 0.01
_K_FRAC = 0.5
_BM = 400
_NSTREAM = 2
_NC, _NS = 2, 16
_NW = _NC * _NS
_HI = jax.lax.Precision.HIGHEST


def _leaky(v):
    return jnp.where(v >= 0.0, v, v * 0.01)


def _main_body(*refs, n_nodes, scale):
    adj_refs = refs[:_NSTREAM]
    (x_ref, w_ref, b_ref, degc_ref, degr_ref, pe_ref, wg_ref, wb_ref,
     bg_ref, bb_ref, wa_ref, wr_ref, out_ref, lb_ref, lf_ref,
     h_vmem) = refs[_NSTREAM:]
    step = pl.program_id(0)

    @pl.when(step == 0)
    def _():
        h_vmem[...] = (jnp.dot(x_ref[...], w_ref[...],
                               preferred_element_type=jnp.float32)
                       + b_ref[...]) * scale

    agg = jnp.concatenate(
        [jnp.dot(a[...], h_vmem[...], preferred_element_type=jnp.float32)
         for a in adj_refs], axis=0)
    degc = degc_ref[...]
    kthr = jnp.sum(degr_ref[...]) * (_K_FRAC / n_nodes)

    pe64 = pe_ref[0:64, :]
    g_mat = jnp.dot(pe64, wg_ref[...], precision=_HI,
                    preferred_element_type=jnp.float32)
    b_mat = jnp.dot(pe64, wb_ref[...], precision=_HI,
                    preferred_element_type=jnp.float32)
    ids = lax.broadcasted_iota(jnp.int32, (degc.shape[0], 64), 1)
    onehot = (ids == degc.astype(jnp.int32)).astype(jnp.float32)
    gamma = _leaky(jnp.dot(onehot, g_mat, precision=_HI,
                           preferred_element_type=jnp.float32) + bg_ref[...])
    beta = _leaky(jnp.dot(onehot, b_mat, precision=_HI,
                          preferred_element_type=jnp.float32) + bb_ref[...])

    deg_safe = jnp.where(degc == 0.0, 1.0, degc)
    ivec = jnp.where(degc == 0.0, 0.0, agg / deg_safe)
    gp1 = gamma + 1.0
    b_add = gp1 * jnp.dot(ivec, wa_ref[...], precision=_HI,
                          preferred_element_type=jnp.float32) + beta
    b_rev = gp1 * jnp.dot(ivec, wr_ref[...], precision=_HI,
                          preferred_element_type=jnp.float32) + beta
    r = (degc < kthr).astype(jnp.float32)
    bias = _OMEGA * (r * b_add - (1.0 - r) * b_rev)
    bm = degc.shape[0]
    h_blk = h_vmem[pl.ds(step * bm, bm), :]
    out_ref[...] = (agg + h_blk + bias) / (degc + 1.0)

    n_add = jnp.sqrt(jnp.sum(b_add * b_add, axis=1, keepdims=True))
    n_rev = jnp.sqrt(jnp.sum(b_rev * b_rev, axis=1, keepdims=True))
    lb_ref[...] = r * n_add + (1.0 - r) * n_rev
    lf_ref[...] = (jnp.sqrt(jnp.sum(gamma * gamma, axis=1, keepdims=True))
                   + jnp.sqrt(jnp.sum(beta * beta, axis=1, keepdims=True)))


def _sc_loss_body(lb_hbm, lf_hbm, idx_hbm, out_lb, out_lf,
                  idx_v, lb_v, lf_v, acc_v, sem, *, b_per_w, n_valid):
    wid = lax.axis_index("s") * _NC + lax.axis_index("c")
    base = wid * b_per_w
    pltpu.sync_copy(idx_hbm.at[pl.ds(base, b_per_w)], idx_v)
    cp_lb = pltpu.async_copy(lb_hbm.at[idx_v], lb_v, sem)
    cp_lf = pltpu.async_copy(lf_hbm.at[idx_v], lf_v, sem)
    cp_lb.wait()
    cp_lf.wait()
    acc_lb = jnp.zeros((16,), jnp.float32)
    acc_lf = jnp.zeros((16,), jnp.float32)
    lanes = lax.iota(jnp.int32, 16)
    for j in range(b_per_w // 16):
        pos = base + (j * 16) + lanes
        mask = pos < n_valid
        acc_lb = acc_lb + jnp.where(mask, lb_v[pl.ds(j * 16, 16)], 0.0)
        acc_lf = acc_lf + jnp.where(mask, lf_v[pl.ds(j * 16, 16)], 0.0)
    acc_v[...] = acc_lb
    pltpu.sync_copy(acc_v, out_lb.at[wid])
    acc_v[...] = acc_lf
    pltpu.sync_copy(acc_v, out_lf.at[wid])


def kernel(x, adj, degree, idx, edge, W, b, W_gamma, W_beta, b_gamma, b_beta,
           W_add, W_rev, PE):
    n, d_in = x.shape
    d_out = W.shape[1]
    dim_m = PE.shape[1]
    scale = float(np.sqrt(dim_m))
    n_idx = idx.shape[0]

    b2 = b.reshape(1, d_out)
    degf = degree.astype(jnp.float32)
    degr = degf.reshape(1, n)

    nblk = n // _BM
    blk = lambda i: (i, 0)
    full = lambda i: (0, 0)
    out, lb_pn, lf_pn = pl.pallas_call(
        functools.partial(_main_body, n_nodes=n, scale=scale),
        grid=(nblk,),
        in_specs=[
            *[pl.BlockSpec((_BM // _NSTREAM, n),
                           functools.partial(
                               lambda i, s: (_NSTREAM * i + s, 0), s=s))
              for s in range(_NSTREAM)],
            pl.BlockSpec((n, d_in), full),
            pl.BlockSpec(W.shape, full),
            pl.BlockSpec((1, d_out), full),
            pl.BlockSpec((_BM, 1), blk),
            pl.BlockSpec((1, n), full),
            pl.BlockSpec(PE.shape, full),
            pl.BlockSpec(W_gamma.shape, full),
            pl.BlockSpec(W_beta.shape, full),
            pl.BlockSpec(b_gamma.shape, full),
            pl.BlockSpec(b_beta.shape, full),
            pl.BlockSpec(W_add.shape, full),
            pl.BlockSpec(W_rev.shape, full),
        ],
        out_specs=[
            pl.BlockSpec((_BM, d_out), blk),
            pl.BlockSpec((_BM, 1), blk),
            pl.BlockSpec((_BM, 1), blk),
        ],
        out_shape=[
            jax.ShapeDtypeStruct((n, d_out), jnp.float32),
            jax.ShapeDtypeStruct((n, 1), jnp.float32),
            jax.ShapeDtypeStruct((n, 1), jnp.float32),
        ],
        scratch_shapes=[pltpu.VMEM((n, d_out), jnp.float32)],
        compiler_params=pltpu.CompilerParams(
            dimension_semantics=("arbitrary",)),
    )(*([adj] * _NSTREAM), x, W, b2, degf, degr, PE, W_gamma, W_beta,
      b_gamma, b_beta, W_add, W_rev)

    b_per_w = -(-n_idx // _NW)
    b_per_w = ((b_per_w + 15) // 16) * 16
    n_pad = b_per_w * _NW
    idx_pad = jnp.concatenate(
        [idx.astype(jnp.int32), jnp.zeros((n_pad - n_idx,), jnp.int32)])

    mesh = plsc.VectorSubcoreMesh(core_axis_name="c", subcore_axis_name="s")
    sc_loss = functools.partial(
        pl.kernel,
        mesh=mesh,
        out_type=[
            jax.ShapeDtypeStruct((_NW, 16), jnp.float32),
            jax.ShapeDtypeStruct((_NW, 16), jnp.float32),
        ],
        scratch_types=[
            pltpu.VMEM((b_per_w,), jnp.int32),
            pltpu.VMEM((b_per_w,), jnp.float32),
            pltpu.VMEM((b_per_w,), jnp.float32),
            pltpu.VMEM((16,), jnp.float32),
            pltpu.SemaphoreType.DMA,
        ],
        compiler_params=pltpu.CompilerParams(needs_layout_passes=False),
    )(functools.partial(_sc_loss_body, b_per_w=b_per_w, n_valid=n_idx))
    plb, plf = sc_loss(lb_pn.reshape(n), lf_pn.reshape(n), idx_pad)

    inv = 1.0 / n_idx
    l_b = jnp.sum(plb) * inv
    l_film = jnp.sum(plf) * inv
    return out, l_b, l_film
